# baseline (speedup 1.0000x reference)
import jax
import jax.numpy as jnp
from jax import lax
from jax.experimental import pallas as pl
from jax.experimental.pallas import tpu as pltpu
from jax.experimental.pallas import tpu_sc as plsc

N = 10000
E = 320000
D = 128
DH = 64
DE = 16

NC = 2
NS = 16
K = 40
EPT = E // NS
NCH = EPT // K
NPAIR = NCH // 2
NP = 10240
RPT = NP // NS


def _prep_body(x_ref, ea_ref, wfc_ref, bfc_ref, wi_ref, wj_ref, w20_ref,
               w21_ref, bb0_ref, bb1_ref, a_ref, t0_ref, t1_ref,
               c0_ref, c1_ref):
    dn = (((1,), (1,)), ((), ()))
    h = lax.dot_general(x_ref[...], wfc_ref[...], dn,
                        preferred_element_type=jnp.float32) + bfc_ref[...]
    a = lax.dot_general(h, wi_ref[...], dn, preferred_element_type=jnp.float32)
    b = lax.dot_general(h, wj_ref[...], dn, preferred_element_type=jnp.float32)
    a_ref[...] = a
    t0_ref[...] = jnp.concatenate([b[:, :DH], h[:, :DH]], axis=1)
    t1_ref[...] = jnp.concatenate([b[:, DH:], h[:, DH:]], axis=1)
    ea3 = ea_ref[...].reshape(-1, 2, DE)
    ea_even = ea3[:, 0, :]
    ea_odd = ea3[:, 1, :]
    w20 = w20_ref[...]
    w21 = w21_ref[...]
    c0_ref[...] = jnp.concatenate(
        [lax.dot_general(ea_even, w20, dn,
                         preferred_element_type=jnp.float32),
         lax.dot_general(ea_odd, w20, dn,
                         preferred_element_type=jnp.float32)],
        axis=1) + bb0_ref[...]
    c1_ref[...] = jnp.concatenate(
        [lax.dot_general(ea_even, w21, dn,
                         preferred_element_type=jnp.float32),
         lax.dot_general(ea_odd, w21, dn,
                         preferred_element_type=jnp.float32)],
        axis=1) + bb1_ref[...]


def _prep(x, edge_attr, w_fc, b_fc, wi, wj, w20, w21, bb0, bb1):
    g = 25
    rn = N // g
    re = E // g
    full = pl.BlockSpec((D, D), lambda i: (0, 0))
    vec = pl.BlockSpec((D,), lambda i: (0,))
    w2spec = pl.BlockSpec((DH, DE), lambda i: (0, 0))
    return pl.pallas_call(
        _prep_body,
        grid=(g,),
        in_specs=[
            pl.BlockSpec((rn, D), lambda i: (i, 0)),
            pl.BlockSpec((re, DE), lambda i: (i, 0)),
            full, vec, full, full,
            w2spec, w2spec, vec, vec,
        ],
        out_specs=[
            pl.BlockSpec((rn, D), lambda i: (i, 0)),
            pl.BlockSpec((rn, D), lambda i: (i, 0)),
            pl.BlockSpec((rn, D), lambda i: (i, 0)),
            pl.BlockSpec((re // 2, D), lambda i: (i, 0)),
            pl.BlockSpec((re // 2, D), lambda i: (i, 0)),
        ],
        out_shape=[
            jax.ShapeDtypeStruct((N, D), jnp.float32),
            jax.ShapeDtypeStruct((N, D), jnp.float32),
            jax.ShapeDtypeStruct((N, D), jnp.float32),
            jax.ShapeDtypeStruct((E // 2, D), jnp.float32),
            jax.ShapeDtypeStruct((E // 2, D), jnp.float32),
        ],
    )(x, edge_attr, w_fc, b_fc, wi, wj, w20, w21, bb0, bb1)


WB = 40


def _sc_edges_body(a_t, t0, t1, c0, c1, src_hbm, dst_hbm,
                   out_hbm,
                   srcb0, dgb0, dsb0, ab0, tb0, cb0, vb0,
                   srcb1, dgb1, dsb1, ab1, tb1, cb1, vb1,
                   acc,
                   sg0, sg1, ss0, ss1, sig0, sig1, sis0, sis1, sc0, sc1):
    cid = lax.axis_index("c")
    sid = lax.axis_index("s")
    srcb = [srcb0, srcb1]
    dgb = [dgb0, dgb1]
    dsb = [dsb0, dsb1]
    ab = [ab0, ab1]
    tb = [tb0, tb1]
    c2b = [cb0, cb1]
    vb = [vb0, vb1]
    sg = [sg0, sg1]
    ss = [ss0, ss1]
    sig = [sig0, sig1]
    sis = [sis0, sis1]
    sc = [sc0, sc1]
    base = sid * EPT
    cbase = sid * (EPT // 2)

    zero16 = jnp.zeros((16,), jnp.float32)

    def _zrow(r, _):
        for g in range(D // 16):
            vb0[r, pl.ds(g * 16, 16)] = zero16
        return 0

    lax.fori_loop(0, K, _zrow, 0)
    for kz in range(RPT // K):
        pltpu.sync_copy(vb0, acc.at[pl.ds(sid * RPT + kz * K, K)])
    plsc.subcore_barrier()

    def issue_gathers(p, c):
        pltpu.async_copy(a_t.at[dgb[p]], ab[p], sg[p])

        @pl.when(cid == 0)
        def _():
            pltpu.async_copy(t0.at[srcb[p]], tb[p], sg[p])

        @pl.when(cid == 1)
        def _():
            pltpu.async_copy(t1.at[srcb[p]], tb[p], sg[p])

    def wait_gathers(p):
        pltpu.make_async_copy(a_t.at[dgb[p]], ab[p], sg[p]).wait()

        @pl.when(cid == 0)
        def _():
            pltpu.make_async_copy(t0.at[srcb[p]], tb[p], sg[p]).wait()

        @pl.when(cid == 1)
        def _():
            pltpu.make_async_copy(t1.at[srcb[p]], tb[p], sg[p]).wait()

    def issue_c(q, pr):
        ro = cbase + pr * K

        @pl.when(cid == 0)
        def _():
            pltpu.async_copy(c0.at[pl.ds(ro, K)], c2b[q], sc[q])

        @pl.when(cid == 1)
        def _():
            pltpu.async_copy(c1.at[pl.ds(ro, K)], c2b[q], sc[q])

    def wait_c(q):
        @pl.when(cid == 0)
        def _():
            pltpu.make_async_copy(c0.at[pl.ds(cbase, K)], c2b[q],
                                  sc[q]).wait()

        @pl.when(cid == 1)
        def _():
            pltpu.make_async_copy(c1.at[pl.ds(cbase, K)], c2b[q],
                                  sc[q]).wait()

    ao = cid * DH

    def compute(p, q):
        def _pair2(e2, _):
            for par in range(2):
                e = 2 * e2 + par
                for g in range(DH // 16):
                    sl = pl.ds(g * 16, 16)
                    sh = pl.ds(DH + g * 16, 16)
                    sa = pl.ds(ao + g * 16, 16)
                    sc_ = pl.ds(par * DH + g * 16, 16)
                    pre = (ab[p][e, sa] + tb[p][e, sl]
                           + c2b[q][(K // 2) * p + e2, sc_])
                    al = jnp.maximum(pre, pre * 0.2)
                    ex = jnp.exp(al)
                    vb[p][e, sl] = ex * tb[p][e, sh]
                    vb[p][e, sh] = ex
            return 0

        lax.fori_loop(0, K // 2, _pair2, 0)

    def half(p, c, i, q):
        pl.when(i > 0)(
            lambda: pltpu.make_async_copy(vb[p], acc.at[dsb[p]], ss[p]).wait())
        pltpu.async_copy(dst_hbm.at[pl.ds(base + c * K, K)], dsb[p], sis[p])
        wait_gathers(p)

        @pl.when(i < NPAIR - 1)
        def _():
            eo2 = base + (c + 2) * K
            pltpu.async_copy(src_hbm.at[pl.ds(eo2, K)], srcb[p], sig[p])
            pltpu.async_copy(dst_hbm.at[pl.ds(eo2, K)], dgb[p], sig[p])

        compute(p, q)

        @pl.when(i < NPAIR - 1)
        def _():
            pltpu.make_async_copy(
                src_hbm.at[pl.ds(base, K)], srcb[p], sig[p]).wait()
            pltpu.make_async_copy(
                dst_hbm.at[pl.ds(base, K)], dgb[p], sig[p]).wait()
            issue_gathers(p, c + 2)

        pltpu.make_async_copy(dst_hbm.at[pl.ds(base, K)], dsb[p], sis[p]).wait()
        pltpu.async_copy(vb[p], acc.at[dsb[p]], ss[p], add=True)

    def do_pair(pr, i, q):
        wait_c(q)
        pl.when(i < NPAIR - 1)(lambda: issue_c(1 - q, pr + 1))
        half(0, 2 * i, i, q)
        half(1, 2 * i + 1, i, q)

    pltpu.sync_copy(src_hbm.at[pl.ds(base, K)], srcb[0])
    pltpu.sync_copy(dst_hbm.at[pl.ds(base, K)], dgb[0])
    pltpu.sync_copy(src_hbm.at[pl.ds(base + K, K)], srcb[1])
    pltpu.sync_copy(dst_hbm.at[pl.ds(base + K, K)], dgb[1])
    issue_gathers(0, 0)
    issue_gathers(1, 1)
    issue_c(0, 0)

    def _quad(i2, _):
        do_pair(2 * i2, 2 * i2, 0)
        do_pair(2 * i2 + 1, 2 * i2 + 1, 1)
        return 0

    lax.fori_loop(0, NPAIR // 2, _quad, 0)

    pltpu.make_async_copy(vb[0], acc.at[dsb[0]], ss[0]).wait()
    pltpu.make_async_copy(vb[1], acc.at[dsb[1]], ss[1]).wait()
    plsc.subcore_barrier()
    r0 = sid * RPT
    pltpu.sync_copy(acc.at[pl.ds(r0, RPT)],
                    out_hbm.at[pl.ds(cid * NP + r0, RPT)])


def _sc_edges(a_t, t0, t1, c0, c1, src, dst):
    mesh = plsc.VectorSubcoreMesh(core_axis_name="c", subcore_axis_name="s",
                                  num_cores=NC, num_subcores=NS)
    buf_set = [
        pltpu.VMEM((K,), jnp.int32),
        pltpu.VMEM((K,), jnp.int32),
        pltpu.VMEM((K,), jnp.int32),
        pltpu.VMEM((K, D), jnp.float32),
        pltpu.VMEM((K, D), jnp.float32),
        pltpu.VMEM((K, D), jnp.float32),
        pltpu.VMEM((K, D), jnp.float32),
    ]
    f = pl.kernel(
        _sc_edges_body,
        out_type=jax.ShapeDtypeStruct((NC * NP, D), jnp.float32),
        mesh=mesh,
        scratch_types=(
            buf_set + buf_set
            + [pltpu.VMEM_SHARED((NP, D), jnp.float32)]
            + [pltpu.SemaphoreType.DMA] * 10
        ),
    )
    return f(a_t, t0, t1, c0, c1, src, dst)


def _combine_body(x_ref, o0_ref, o1_ref, out_ref):
    m = jnp.concatenate([o0_ref[0, :, :DH], o1_ref[0, :, :DH]], axis=1)
    s = jnp.concatenate([o0_ref[0, :, DH:], o1_ref[0, :, DH:]], axis=1)
    out_ref[...] = x_ref[...] + m / (s + 1e-16)


def _combine(x, sc_out):
    r = 2000
    nb = N // r
    sc3 = sc_out.reshape(NC, NP, D)
    return pl.pallas_call(
        _combine_body,
        grid=(nb,),
        in_specs=[
            pl.BlockSpec((r, D), lambda i: (i, 0)),
            pl.BlockSpec((1, r, D), lambda i: (0, i, 0)),
            pl.BlockSpec((1, r, D), lambda i: (1, i, 0)),
        ],
        out_specs=pl.BlockSpec((r, D), lambda i: (i, 0)),
        out_shape=jax.ShapeDtypeStruct((N, D), jnp.float32),
    )(x, sc3, sc3)


@jax.jit
def kernel(x, edge_index, edge_attr, W_fc, b_fc, W_attn, b_attn):
    ei = edge_index.astype(jnp.int32)
    src = ei[0]
    dst = ei[1]
    wi = W_attn[:, :D]
    wj = W_attn[:, D:2 * D]
    we = W_attn[:, 2 * D:]

    w20 = we[:DH]
    w21 = we[DH:]
    bb0 = jnp.concatenate([b_attn[:DH], b_attn[:DH]])
    bb1 = jnp.concatenate([b_attn[DH:], b_attn[DH:]])

    a_t, t0, t1, c0, c1 = _prep(x, edge_attr, W_fc, b_fc, wi, wj,
                                w20, w21, bb0, bb1)
    sc_out = _sc_edges(a_t, t0, t1, c0, c1, src, dst)
    return _combine(x, sc_out)


# speedup vs baseline: 2.0095x; 2.0095x over previous
import jax
import jax.numpy as jnp
from jax import lax
from jax.experimental import pallas as pl
from jax.experimental.pallas import tpu as pltpu
from jax.experimental.pallas import tpu_sc as plsc

N = 10000
E = 320000
D = 128
DH = 64
DE = 16

NC = 2
NS = 16
K = 40
EPT = E // NS
NCH = EPT // K
NPAIR = NCH // 2
NP = 10240
RPT = NP // NS


def _prep_body(x_ref, ea_ref, wfc_ref, bfc_ref, wi_ref, wj_ref, we_ref,
               ba_ref, a_ref, t0_ref, t1_ref, c0_ref, c1_ref):
    dn = (((1,), (1,)), ((), ()))
    h = lax.dot_general(x_ref[...], wfc_ref[...], dn,
                        preferred_element_type=jnp.float32) + bfc_ref[...]
    a = lax.dot_general(h, wi_ref[...], dn, preferred_element_type=jnp.float32)
    b = lax.dot_general(h, wj_ref[...], dn, preferred_element_type=jnp.float32)
    a_ref[...] = a
    t0_ref[...] = jnp.concatenate([b[:, :DH], h[:, :DH]], axis=1)
    t1_ref[...] = jnp.concatenate([b[:, DH:], h[:, DH:]], axis=1)
    c = lax.dot_general(ea_ref[...], we_ref[...], dn,
                        preferred_element_type=jnp.float32) + ba_ref[...]
    c0_ref[...] = c[:, :DH]
    c1_ref[...] = c[:, DH:]


def _prep(x, edge_attr, w_fc, b_fc, wi, wj, we, b_attn):
    g = 25
    rn = N // g
    re = E // g
    full = pl.BlockSpec((D, D), lambda i: (0, 0))
    vec = pl.BlockSpec((D,), lambda i: (0,))
    return pl.pallas_call(
        _prep_body,
        grid=(g,),
        in_specs=[
            pl.BlockSpec((rn, D), lambda i: (i, 0)),
            pl.BlockSpec((re, DE), lambda i: (i, 0)),
            full, vec, full, full,
            pl.BlockSpec((D, DE), lambda i: (0, 0)),
            vec,
        ],
        out_specs=[
            pl.BlockSpec((rn, D), lambda i: (i, 0)),
            pl.BlockSpec((rn, D), lambda i: (i, 0)),
            pl.BlockSpec((rn, D), lambda i: (i, 0)),
            pl.BlockSpec((re, DH), lambda i: (i, 0)),
            pl.BlockSpec((re, DH), lambda i: (i, 0)),
        ],
        out_shape=[
            jax.ShapeDtypeStruct((N, D), jnp.float32),
            jax.ShapeDtypeStruct((N, D), jnp.float32),
            jax.ShapeDtypeStruct((N, D), jnp.float32),
            jax.ShapeDtypeStruct((E, DH), jnp.float32),
            jax.ShapeDtypeStruct((E, DH), jnp.float32),
        ],
    )(x, edge_attr, w_fc, b_fc, wi, wj, we, b_attn)


WB = 40


def _sc_edges_body(a_t, t0, t1, c0, c1, src_hbm, dst_hbm,
                   out_hbm,
                   srcb0, dgb0, dsb0, ab0, tb0, cb0, vb0,
                   srcb1, dgb1, dsb1, ab1, tb1, cb1, vb1,
                   acc,
                   sg0, sg1, ss0, ss1, sig0, sig1, sis0, sis1):
    cid = lax.axis_index("c")
    sid = lax.axis_index("s")
    srcb = [srcb0, srcb1]
    dgb = [dgb0, dgb1]
    dsb = [dsb0, dsb1]
    ab = [ab0, ab1]
    tb = [tb0, tb1]
    cb = [cb0, cb1]
    vb = [vb0, vb1]
    sg = [sg0, sg1]
    ss = [ss0, ss1]
    sig = [sig0, sig1]
    sis = [sis0, sis1]
    base = sid * EPT

    zero16 = jnp.zeros((16,), jnp.float32)

    def _zrow(r, _):
        for g in range(D // 16):
            vb0[r, pl.ds(g * 16, 16)] = zero16
        return 0

    lax.fori_loop(0, K, _zrow, 0)
    for kz in range(RPT // K):
        pltpu.sync_copy(vb0, acc.at[pl.ds(sid * RPT + kz * K, K)])
    plsc.subcore_barrier()

    def issue_gathers(p, c):
        eo = base + c * K
        pltpu.async_copy(a_t.at[dgb[p]], ab[p], sg[p])

        @pl.when(cid == 0)
        def _():
            pltpu.async_copy(t0.at[srcb[p]], tb[p], sg[p])
            pltpu.async_copy(c0.at[pl.ds(eo, K)], cb[p], sg[p])

        @pl.when(cid == 1)
        def _():
            pltpu.async_copy(t1.at[srcb[p]], tb[p], sg[p])
            pltpu.async_copy(c1.at[pl.ds(eo, K)], cb[p], sg[p])

    def wait_gathers(p):
        pltpu.make_async_copy(a_t.at[dgb[p]], ab[p], sg[p]).wait()

        @pl.when(cid == 0)
        def _():
            pltpu.make_async_copy(t0.at[srcb[p]], tb[p], sg[p]).wait()
            pltpu.make_async_copy(c0.at[pl.ds(base, K)], cb[p], sg[p]).wait()

        @pl.when(cid == 1)
        def _():
            pltpu.make_async_copy(t1.at[srcb[p]], tb[p], sg[p]).wait()
            pltpu.make_async_copy(c1.at[pl.ds(base, K)], cb[p], sg[p]).wait()

    def compute(p):
        def go(ao):
            def _edge(e, _):
                for g in range(DH // 16):
                    sl = pl.ds(g * 16, 16)
                    sh = pl.ds(DH + g * 16, 16)
                    sa = pl.ds(ao + g * 16, 16)
                    pre = ab[p][e, sa] + tb[p][e, sl] + cb[p][e, sl]
                    al = jnp.maximum(pre, pre * 0.2)
                    ex = jnp.exp(al)
                    vb[p][e, sl] = ex * tb[p][e, sh]
                    vb[p][e, sh] = ex
                return 0

            lax.fori_loop(0, K, _edge, 0)

        pl.when(cid == 0)(lambda: go(0))
        pl.when(cid == 1)(lambda: go(DH))

    def half(p, c, i):
        pl.when(i > 0)(
            lambda: pltpu.make_async_copy(vb[p], acc.at[dsb[p]], ss[p]).wait())
        pltpu.async_copy(dst_hbm.at[pl.ds(base + c * K, K)], dsb[p], sis[p])
        wait_gathers(p)

        @pl.when(i < NPAIR - 1)
        def _():
            eo2 = base + (c + 2) * K
            pltpu.async_copy(src_hbm.at[pl.ds(eo2, K)], srcb[p], sig[p])
            pltpu.async_copy(dst_hbm.at[pl.ds(eo2, K)], dgb[p], sig[p])

        compute(p)

        @pl.when(i < NPAIR - 1)
        def _():
            pltpu.make_async_copy(
                src_hbm.at[pl.ds(base, K)], srcb[p], sig[p]).wait()
            pltpu.make_async_copy(
                dst_hbm.at[pl.ds(base, K)], dgb[p], sig[p]).wait()
            issue_gathers(p, c + 2)

        pltpu.make_async_copy(dst_hbm.at[pl.ds(base, K)], dsb[p], sis[p]).wait()
        pltpu.async_copy(vb[p], acc.at[dsb[p]], ss[p], add=True)

    pltpu.sync_copy(src_hbm.at[pl.ds(base, K)], srcb[0])
    pltpu.sync_copy(dst_hbm.at[pl.ds(base, K)], dgb[0])
    pltpu.sync_copy(src_hbm.at[pl.ds(base + K, K)], srcb[1])
    pltpu.sync_copy(dst_hbm.at[pl.ds(base + K, K)], dgb[1])
    issue_gathers(0, 0)
    issue_gathers(1, 1)

    def _pair(i, _):
        half(0, 2 * i, i)
        half(1, 2 * i + 1, i)
        return 0

    lax.fori_loop(0, NPAIR, _pair, 0)

    pltpu.make_async_copy(vb[0], acc.at[dsb[0]], ss[0]).wait()
    pltpu.make_async_copy(vb[1], acc.at[dsb[1]], ss[1]).wait()
    plsc.subcore_barrier()
    r0 = sid * RPT
    pltpu.sync_copy(acc.at[pl.ds(r0, RPT)],
                    out_hbm.at[pl.ds(cid * NP + r0, RPT)])


def _sc_edges(a_t, t0, t1, c0, c1, src, dst):
    mesh = plsc.VectorSubcoreMesh(core_axis_name="c", subcore_axis_name="s",
                                  num_cores=NC, num_subcores=NS)
    buf_set = [
        pltpu.VMEM((K,), jnp.int32),
        pltpu.VMEM((K,), jnp.int32),
        pltpu.VMEM((K,), jnp.int32),
        pltpu.VMEM((K, D), jnp.float32),
        pltpu.VMEM((K, D), jnp.float32),
        pltpu.VMEM((K, DH), jnp.float32),
        pltpu.VMEM((K, D), jnp.float32),
    ]
    f = pl.kernel(
        _sc_edges_body,
        out_type=jax.ShapeDtypeStruct((NC * NP, D), jnp.float32),
        mesh=mesh,
        scratch_types=(
            buf_set + buf_set
            + [pltpu.VMEM_SHARED((NP, D), jnp.float32)]
            + [pltpu.SemaphoreType.DMA] * 8
        ),
    )
    return f(a_t, t0, t1, c0, c1, src, dst)


def _combine_body(x_ref, o0_ref, o1_ref, out_ref):
    m = jnp.concatenate([o0_ref[0, :, :DH], o1_ref[0, :, :DH]], axis=1)
    s = jnp.concatenate([o0_ref[0, :, DH:], o1_ref[0, :, DH:]], axis=1)
    out_ref[...] = x_ref[...] + m / (s + 1e-16)


def _combine(x, sc_out):
    r = 2000
    nb = N // r
    sc3 = sc_out.reshape(NC, NP, D)
    return pl.pallas_call(
        _combine_body,
        grid=(nb,),
        in_specs=[
            pl.BlockSpec((r, D), lambda i: (i, 0)),
            pl.BlockSpec((1, r, D), lambda i: (0, i, 0)),
            pl.BlockSpec((1, r, D), lambda i: (1, i, 0)),
        ],
        out_specs=pl.BlockSpec((r, D), lambda i: (i, 0)),
        out_shape=jax.ShapeDtypeStruct((N, D), jnp.float32),
    )(x, sc3, sc3)


@jax.jit
def kernel(x, edge_index, edge_attr, W_fc, b_fc, W_attn, b_attn):
    ei = edge_index.astype(jnp.int32)
    src = ei[0]
    dst = ei[1]
    wi = W_attn[:, :D]
    wj = W_attn[:, D:2 * D]
    we = W_attn[:, 2 * D:]

    a_t, t0, t1, c0, c1 = _prep(x, edge_attr, W_fc, b_fc, wi, wj, we, b_attn)
    sc_out = _sc_edges(a_t, t0, t1, c0, c1, src, dst)
    return _combine(x, sc_out)
